# baseline (speedup 1.0000x reference)
import jax
import jax.numpy as jnp
from jax.experimental import pallas as pl
from jax.experimental.pallas import tpu as pltpu

_BLK = 16384
_K = 512
_KC = 64
_D = 3


def _vq_body(xt_ref, m_ref, cb_ref, cbm2_ref, cbt_ref, scale_ref, qc_ref,
             idx_ref, loss_ref, acc_ref, cnt_ref, csq_ref):
    i = pl.program_id(0)
    nsteps = pl.num_programs(0)

    xt = xt_ref[...]
    sc = scale_ref[...]
    flat = xt * sc

    fx = flat[0:1, :]
    fy = flat[1:2, :]
    fz = flat[2:3, :]
    fsq = fx * fx + fy * fy + fz * fz

    @pl.when(i == 0)
    def _csq():
        cb = cb_ref[...]
        csq_ref[...] = jnp.sum(cb * cb, axis=1, keepdims=True)

    csq = csq_ref[...]

    cmins = []
    cidxs = []
    for c in range(_K // _KC):
        lo = c * _KC
        cross_c = jax.lax.dot_general(
            cbm2_ref[lo:lo + _KC, :], flat, (((1,), (0,)), ((), ())),
            preferred_element_type=jnp.float32)
        dc = fsq + cross_c + csq[lo:lo + _KC, :]
        cmin = jnp.min(dc, axis=0, keepdims=True)
        iota_c = lo + jax.lax.broadcasted_iota(jnp.int32, (_KC, _BLK), 0)
        cidx = jnp.min(jnp.where(dc == cmin, iota_c, _K), axis=0,
                       keepdims=True)
        cmins.append(cmin)
        cidxs.append(cidx)

    cmins = jnp.concatenate(cmins, axis=0)
    cidxs = jnp.concatenate(cidxs, axis=0)
    minv = jnp.min(cmins, axis=0, keepdims=True)
    idx = jnp.min(jnp.where(cmins == minv, cidxs, _K), axis=0,
                  keepdims=True)

    iota = jax.lax.broadcasted_iota(jnp.int32, (_K, _BLK), 0)

    onehot = (iota == idx).astype(jnp.float32)
    quant = jax.lax.dot_general(cbt_ref[...], onehot, (((1,), (0,)), ((), ())),
                                preferred_element_type=jnp.float32)

    sabs = jnp.abs(xt[0:1, :]) + jnp.abs(xt[1:2, :]) + jnp.abs(xt[2:3, :])
    valid = (sabs > 0.0) & m_ref[...]
    vm = valid.astype(jnp.float32)

    diff = flat - quant
    part = jnp.sum((diff * diff) * vm)

    @pl.when(i == 0)
    def _init():
        acc_ref[0, 0] = 0.0
        cnt_ref[0, 0] = 0.0

    acc_ref[0, 0] += part
    cnt_ref[0, 0] += jnp.sum(vm)

    q_st = flat + (quant - flat)
    qc_ref[...] = jnp.where(valid, q_st, flat) / sc
    idx_ref[...] = jnp.where(valid, idx, 0)

    @pl.when(i == nsteps - 1)
    def _fin():
        n = jnp.maximum(cnt_ref[0, 0], 1.0)
        val = 0.25 * (acc_ref[0, 0] / (n * float(_D)))
        loss_ref[...] = jnp.full((1, 1), val, jnp.float32)


def kernel(coordinates, attention_mask, codebook, coord_scale):
    B, S, D = coordinates.shape
    N = B * S
    xt = coordinates.reshape(N, D).T
    m = attention_mask.reshape(1, N)
    scale = coord_scale.reshape(D, 1)

    grid = (N // _BLK,)
    qc, idx, loss = pl.pallas_call(
        _vq_body,
        grid=grid,
        in_specs=[
            pl.BlockSpec((D, _BLK), lambda i: (0, i)),
            pl.BlockSpec((1, _BLK), lambda i: (0, i)),
            pl.BlockSpec((_K, D), lambda i: (0, 0)),
            pl.BlockSpec((_K, D), lambda i: (0, 0)),
            pl.BlockSpec((D, _K), lambda i: (0, 0)),
            pl.BlockSpec((D, 1), lambda i: (0, 0)),
        ],
        out_specs=[
            pl.BlockSpec((D, _BLK), lambda i: (0, i)),
            pl.BlockSpec((1, _BLK), lambda i: (0, i)),
            pl.BlockSpec((1, 1), lambda i: (0, 0)),
        ],
        out_shape=[
            jax.ShapeDtypeStruct((D, N), jnp.float32),
            jax.ShapeDtypeStruct((1, N), jnp.int32),
            jax.ShapeDtypeStruct((1, 1), jnp.float32),
        ],
        scratch_shapes=[
            pltpu.SMEM((1, 1), jnp.float32),
            pltpu.SMEM((1, 1), jnp.float32),
            pltpu.VMEM((_K, 1), jnp.float32),
        ],
    )(xt, m, codebook, codebook * (-2.0), codebook.T, scale)

    quantized_coords = qc.T.reshape(B, S, D)
    vq_loss = loss[0, 0]
    indices = idx.reshape(B, S)
    return quantized_coords, vq_loss, indices


# speedup vs baseline: 1.2319x; 1.2319x over previous
import jax
import jax.numpy as jnp
from jax.experimental import pallas as pl
from jax.experimental.pallas import tpu as pltpu

_BLK = 16384
_K = 512
_D = 3


def _vq_body(xt_ref, m_ref, cb_ref, cbm2_ref, cbt1_ref, scale_ref, qc_ref,
             idx_ref, loss_ref, acc_ref, cnt_ref, csq_ref):
    i = pl.program_id(0)
    nsteps = pl.num_programs(0)

    xt = xt_ref[...]
    sc = scale_ref[...]
    flat = xt * sc

    fx = flat[0:1, :]
    fy = flat[1:2, :]
    fz = flat[2:3, :]
    fsq = fx * fx + fy * fy + fz * fz

    @pl.when(i == 0)
    def _csq():
        cb = cb_ref[...]
        csq_ref[...] = jnp.sum(cb * cb, axis=1, keepdims=True)

    csq = csq_ref[...]

    cross2 = jax.lax.dot_general(cbm2_ref[...], flat, (((1,), (0,)), ((), ())),
                                 preferred_element_type=jnp.float32)
    dist = fsq + cross2 + csq

    accd = dist[0:8, :]
    acci = jnp.zeros((8, _BLK), jnp.int32)
    for r in range(1, _K // 8):
        dr = dist[8 * r:8 * r + 8, :]
        take = accd <= dr
        accd = jnp.where(take, accd, dr)
        acci = jnp.where(take, acci, r)
    iota8 = jax.lax.broadcasted_iota(jnp.int32, (8, _BLK), 0)
    accf = acci * 8 + iota8
    for h in (4, 2, 1):
        dA, dB = accd[0:h, :], accd[h:2 * h, :]
        iA, iB = accf[0:h, :], accf[h:2 * h, :]
        take = (dA < dB) | ((dA == dB) & (iA < iB))
        accd = jnp.where(take, dA, dB)
        accf = jnp.where(take, iA, iB)
    minv = accd
    idx = accf

    multi = (dist == minv).astype(jnp.float32)
    qn = jax.lax.dot_general(cbt1_ref[...], multi, (((1,), (0,)), ((), ())),
                             preferred_element_type=jnp.float32)
    quant = qn[0:3, :] / qn[3:4, :]

    sabs = jnp.abs(xt[0:1, :]) + jnp.abs(xt[1:2, :]) + jnp.abs(xt[2:3, :])
    valid = (sabs > 0.0) & m_ref[...]
    vm = valid.astype(jnp.float32)

    diff = flat - quant
    part = jnp.sum((diff * diff) * vm)

    @pl.when(i == 0)
    def _init():
        acc_ref[0, 0] = 0.0
        cnt_ref[0, 0] = 0.0

    acc_ref[0, 0] += part
    cnt_ref[0, 0] += jnp.sum(vm)

    q_st = flat + (quant - flat)
    qc_ref[...] = jnp.where(valid, q_st, flat) / sc
    idx_ref[...] = jnp.where(valid, idx, 0)

    @pl.when(i == nsteps - 1)
    def _fin():
        n = jnp.maximum(cnt_ref[0, 0], 1.0)
        val = 0.25 * (acc_ref[0, 0] / (n * float(_D)))
        loss_ref[...] = jnp.full((1, 1), val, jnp.float32)


def kernel(coordinates, attention_mask, codebook, coord_scale):
    B, S, D = coordinates.shape
    N = B * S
    xt = coordinates.reshape(N, D).T
    m = attention_mask.reshape(1, N)
    scale = coord_scale.reshape(D, 1)

    grid = (N // _BLK,)
    qc, idx, loss = pl.pallas_call(
        _vq_body,
        grid=grid,
        in_specs=[
            pl.BlockSpec((D, _BLK), lambda i: (0, i)),
            pl.BlockSpec((1, _BLK), lambda i: (0, i)),
            pl.BlockSpec((_K, D), lambda i: (0, 0)),
            pl.BlockSpec((_K, D), lambda i: (0, 0)),
            pl.BlockSpec((D + 1, _K), lambda i: (0, 0)),
            pl.BlockSpec((D, 1), lambda i: (0, 0)),
        ],
        out_specs=[
            pl.BlockSpec((D, _BLK), lambda i: (0, i)),
            pl.BlockSpec((1, _BLK), lambda i: (0, i)),
            pl.BlockSpec((1, 1), lambda i: (0, 0)),
        ],
        out_shape=[
            jax.ShapeDtypeStruct((D, N), jnp.float32),
            jax.ShapeDtypeStruct((1, N), jnp.int32),
            jax.ShapeDtypeStruct((1, 1), jnp.float32),
        ],
        scratch_shapes=[
            pltpu.SMEM((1, 1), jnp.float32),
            pltpu.SMEM((1, 1), jnp.float32),
            pltpu.VMEM((_K, 1), jnp.float32),
        ],
    )(xt, m, codebook, codebook * (-2.0),
      jnp.concatenate([codebook.T, jnp.ones((1, _K), jnp.float32)], axis=0),
      scale)

    quantized_coords = qc.T.reshape(B, S, D)
    vq_loss = loss[0, 0]
    indices = idx.reshape(B, S)
    return quantized_coords, vq_loss, indices


# speedup vs baseline: 1.3035x; 1.0581x over previous
import jax
import jax.numpy as jnp
from jax.experimental import pallas as pl
from jax.experimental.pallas import tpu as pltpu

_BLK = 16384
_K = 512
_D = 3


def _vq_body(xt_ref, m_ref, cb_ref, cbm2_ref, cbt1_ref, scale_ref, qc_ref,
             idx_ref, loss_ref, acc_ref, cnt_ref, csq_ref):
    i = pl.program_id(0)
    nsteps = pl.num_programs(0)

    xt = xt_ref[...]
    sc = scale_ref[...]
    flat = xt * sc

    fx = flat[0:1, :]
    fy = flat[1:2, :]
    fz = flat[2:3, :]
    fsq = fx * fx + fy * fy + fz * fz

    @pl.when(i == 0)
    def _csq():
        cb = cb_ref[...]
        csq_ref[...] = jnp.sum(cb * cb, axis=1, keepdims=True)

    csq = csq_ref[...]

    cross2 = jax.lax.dot_general(cbm2_ref[...], flat, (((1,), (0,)), ((), ())),
                                 preferred_element_type=jnp.float32)
    dist = fsq + cross2 + csq

    accd = dist[0:8, :]
    acci = jnp.zeros((8, _BLK), jnp.int32)
    for r in range(1, _K // 8):
        dr = dist[8 * r:8 * r + 8, :]
        take = accd <= dr
        accd = jnp.where(take, accd, dr)
        acci = jnp.where(take, acci, r)
    iota8 = jax.lax.broadcasted_iota(jnp.int32, (8, _BLK), 0)
    accf = acci * 8 + iota8
    for h in (4, 2, 1):
        dA, dB = accd[0:h, :], accd[h:2 * h, :]
        iA, iB = accf[0:h, :], accf[h:2 * h, :]
        take = (dA < dB) | ((dA == dB) & (iA < iB))
        accd = jnp.where(take, dA, dB)
        accf = jnp.where(take, iA, iB)
    minv = accd
    idx = accf

    iota = jax.lax.broadcasted_iota(jnp.int32, (_K, _BLK), 0)
    onehot = (iota == idx).astype(jnp.float32)
    quant = jax.lax.dot_general(cbt1_ref[0:3, :], onehot,
                                (((1,), (0,)), ((), ())),
                                preferred_element_type=jnp.float32)

    sabs = jnp.abs(xt[0:1, :]) + jnp.abs(xt[1:2, :]) + jnp.abs(xt[2:3, :])
    valid = (sabs > 0.0) & m_ref[...]
    vm = valid.astype(jnp.float32)

    diff = flat - quant
    part = jnp.sum((diff * diff) * vm)

    @pl.when(i == 0)
    def _init():
        acc_ref[0, 0] = 0.0
        cnt_ref[0, 0] = 0.0

    acc_ref[0, 0] += part
    cnt_ref[0, 0] += jnp.sum(vm)

    q_st = flat + (quant - flat)
    qc_ref[...] = jnp.where(valid, q_st, flat) / sc
    idx_ref[...] = jnp.where(valid, idx, 0)

    @pl.when(i == nsteps - 1)
    def _fin():
        n = jnp.maximum(cnt_ref[0, 0], 1.0)
        val = 0.25 * (acc_ref[0, 0] / (n * float(_D)))
        loss_ref[...] = jnp.full((1, 1), val, jnp.float32)


def kernel(coordinates, attention_mask, codebook, coord_scale):
    B, S, D = coordinates.shape
    N = B * S
    xt = coordinates.reshape(N, D).T
    m = attention_mask.reshape(1, N)
    scale = coord_scale.reshape(D, 1)

    grid = (N // _BLK,)
    qc, idx, loss = pl.pallas_call(
        _vq_body,
        grid=grid,
        in_specs=[
            pl.BlockSpec((D, _BLK), lambda i: (0, i)),
            pl.BlockSpec((1, _BLK), lambda i: (0, i)),
            pl.BlockSpec((_K, D), lambda i: (0, 0)),
            pl.BlockSpec((_K, D), lambda i: (0, 0)),
            pl.BlockSpec((D + 1, _K), lambda i: (0, 0)),
            pl.BlockSpec((D, 1), lambda i: (0, 0)),
        ],
        out_specs=[
            pl.BlockSpec((D, _BLK), lambda i: (0, i)),
            pl.BlockSpec((1, _BLK), lambda i: (0, i)),
            pl.BlockSpec((1, 1), lambda i: (0, 0)),
        ],
        out_shape=[
            jax.ShapeDtypeStruct((D, N), jnp.float32),
            jax.ShapeDtypeStruct((1, N), jnp.int32),
            jax.ShapeDtypeStruct((1, 1), jnp.float32),
        ],
        scratch_shapes=[
            pltpu.SMEM((1, 1), jnp.float32),
            pltpu.SMEM((1, 1), jnp.float32),
            pltpu.VMEM((_K, 1), jnp.float32),
        ],
    )(xt, m, codebook, codebook * (-2.0),
      jnp.concatenate([codebook.T, jnp.ones((1, _K), jnp.float32)], axis=0),
      scale)

    quantized_coords = qc.T.reshape(B, S, D)
    vq_loss = loss[0, 0]
    indices = idx.reshape(B, S)
    return quantized_coords, vq_loss, indices


# speedup vs baseline: 1.3207x; 1.0132x over previous
import jax
import jax.numpy as jnp
from jax.experimental import pallas as pl
from jax.experimental.pallas import tpu as pltpu

_BLK = 8192
_K = 512
_D = 3


def _vq_body(xt_ref, m_ref, cb_ref, cbm2_ref, cbt1_ref, scale_ref, qc_ref,
             idx_ref, loss_ref, acc_ref, cnt_ref, csq_ref):
    i = pl.program_id(0)
    nsteps = pl.num_programs(0)

    xt = xt_ref[...]
    sc = scale_ref[...]
    flat = xt * sc

    fx = flat[0:1, :]
    fy = flat[1:2, :]
    fz = flat[2:3, :]
    fsq = fx * fx + fy * fy + fz * fz

    @pl.when(i == 0)
    def _csq():
        cb = cb_ref[...]
        csq_ref[...] = jnp.sum(cb * cb, axis=1, keepdims=True)

    csq = csq_ref[...]

    cross2 = jax.lax.dot_general(cbm2_ref[...], flat, (((1,), (0,)), ((), ())),
                                 preferred_element_type=jnp.float32)
    dist = fsq + cross2 + csq

    accd = dist[0:8, :]
    acci = jnp.zeros((8, _BLK), jnp.int32)
    for r in range(1, _K // 8):
        dr = dist[8 * r:8 * r + 8, :]
        take = accd <= dr
        accd = jnp.where(take, accd, dr)
        acci = jnp.where(take, acci, r)
    iota8 = jax.lax.broadcasted_iota(jnp.int32, (8, _BLK), 0)
    accf = acci * 8 + iota8
    for h in (4, 2, 1):
        dA, dB = accd[0:h, :], accd[h:2 * h, :]
        iA, iB = accf[0:h, :], accf[h:2 * h, :]
        take = (dA < dB) | ((dA == dB) & (iA < iB))
        accd = jnp.where(take, dA, dB)
        accf = jnp.where(take, iA, iB)
    minv = accd
    idx = accf

    iota = jax.lax.broadcasted_iota(jnp.int32, (_K, _BLK), 0)
    onehot = (iota == idx).astype(jnp.float32)
    quant = jax.lax.dot_general(cbt1_ref[0:3, :], onehot,
                                (((1,), (0,)), ((), ())),
                                preferred_element_type=jnp.float32)

    sabs = jnp.abs(xt[0:1, :]) + jnp.abs(xt[1:2, :]) + jnp.abs(xt[2:3, :])
    valid = (sabs > 0.0) & m_ref[...]
    vm = valid.astype(jnp.float32)

    diff = flat - quant
    part = jnp.sum((diff * diff) * vm)

    @pl.when(i == 0)
    def _init():
        acc_ref[0, 0] = 0.0
        cnt_ref[0, 0] = 0.0

    acc_ref[0, 0] += part
    cnt_ref[0, 0] += jnp.sum(vm)

    q_st = flat + (quant - flat)
    qc_ref[...] = jnp.where(valid, q_st, flat) / sc
    idx_ref[...] = jnp.where(valid, idx, 0)

    @pl.when(i == nsteps - 1)
    def _fin():
        n = jnp.maximum(cnt_ref[0, 0], 1.0)
        val = 0.25 * (acc_ref[0, 0] / (n * float(_D)))
        loss_ref[...] = jnp.full((1, 1), val, jnp.float32)


def kernel(coordinates, attention_mask, codebook, coord_scale):
    B, S, D = coordinates.shape
    N = B * S
    xt = coordinates.reshape(N, D).T
    m = attention_mask.reshape(1, N)
    scale = coord_scale.reshape(D, 1)

    grid = (N // _BLK,)
    qc, idx, loss = pl.pallas_call(
        _vq_body,
        grid=grid,
        in_specs=[
            pl.BlockSpec((D, _BLK), lambda i: (0, i)),
            pl.BlockSpec((1, _BLK), lambda i: (0, i)),
            pl.BlockSpec((_K, D), lambda i: (0, 0)),
            pl.BlockSpec((_K, D), lambda i: (0, 0)),
            pl.BlockSpec((D + 1, _K), lambda i: (0, 0)),
            pl.BlockSpec((D, 1), lambda i: (0, 0)),
        ],
        out_specs=[
            pl.BlockSpec((D, _BLK), lambda i: (0, i)),
            pl.BlockSpec((1, _BLK), lambda i: (0, i)),
            pl.BlockSpec((1, 1), lambda i: (0, 0)),
        ],
        out_shape=[
            jax.ShapeDtypeStruct((D, N), jnp.float32),
            jax.ShapeDtypeStruct((1, N), jnp.int32),
            jax.ShapeDtypeStruct((1, 1), jnp.float32),
        ],
        scratch_shapes=[
            pltpu.SMEM((1, 1), jnp.float32),
            pltpu.SMEM((1, 1), jnp.float32),
            pltpu.VMEM((_K, 1), jnp.float32),
        ],
    )(xt, m, codebook, codebook * (-2.0),
      jnp.concatenate([codebook.T, jnp.ones((1, _K), jnp.float32)], axis=0),
      scale)

    quantized_coords = qc.T.reshape(B, S, D)
    vq_loss = loss[0, 0]
    indices = idx.reshape(B, S)
    return quantized_coords, vq_loss, indices
